# baseline (speedup 1.0000x reference)
import numpy as np

import jax
import jax.numpy as jnp
from jax.experimental import pallas as pl
from jax.experimental.pallas import tpu as pltpu
from jax.sharding import Mesh, PartitionSpec as P

_ALPHA = 5.0
_LAM = 1.0
_BR = 8
_L2E = 1.4426950408889634
_LN2 = 0.6931471805599453


def _dtsh_body(u_loc_ref, y_loc_ref, u_ref, y_ref, out_ref):
    i = pl.program_id(0)

    u_blk = u_loc_ref[pl.ds(i * _BR, _BR), :]
    y_blk = y_loc_ref[pl.ds(i * _BR, _BR), :]

    ip = jax.lax.dot_general(
        u_blk, u_ref[...], (((1,), (1,)), ((), ())),
        preferred_element_type=jnp.float32,
        precision=jax.lax.Precision.HIGHEST,
    )
    sim = jax.lax.dot_general(
        y_blk, y_ref[...], (((1,), (1,)), ((), ())),
        preferred_element_type=jnp.float32,
        precision=jax.lax.Precision.HIGHEST,
    )
    pos = (sim > 0).astype(jnp.float32)
    neg = 1.0 - pos
    npos = jnp.sum(pos, axis=1)
    nneg = jnp.sum(neg, axis=1)

    a = ip * _L2E
    c = a + (_ALPHA * _L2E)
    tp = a[:, :, None] - c[:, None, :]
    tc = jnp.clip(tp, -100.0 * _L2E, 50.0 * _L2E)
    g = jnp.log2(1.0 + jnp.exp2(tc))
    f = g - tc

    fp = jnp.sum(f * pos[:, :, None], axis=1)
    num = jnp.sum(fp * neg, axis=1) * _LN2
    pair_count = jnp.maximum(npos * nneg, 1.0)
    row_loss = num / pair_count
    valid = (npos > 0.0) & (nneg > 0.0)
    contrib = jnp.sum(jnp.where(valid, row_loss, 0.0))
    vcount = jnp.sum(valid.astype(jnp.float32))

    q = jnp.sum((u_blk - jnp.sign(u_blk)) ** 2)

    lane = jax.lax.broadcasted_iota(jnp.int32, (1, 1, 128), 2)
    vals = jnp.where(
        lane == 0, contrib,
        jnp.where(lane == 1, vcount, jnp.where(lane == 2, q, 0.0)))
    out_ref[...] = vals


def _row_block_call(u_loc, y_loc, u_all, y_all):
    n_loc, bit = u_loc.shape
    n, c_pad = y_all.shape
    g_loc = n_loc // _BR
    return pl.pallas_call(
        _dtsh_body,
        out_shape=jax.ShapeDtypeStruct((g_loc, 1, 128), jnp.float32),
        grid=(g_loc,),
        in_specs=[
            pl.BlockSpec((n_loc, bit), lambda i: (0, 0)),
            pl.BlockSpec((n_loc, c_pad), lambda i: (0, 0)),
            pl.BlockSpec((n, bit), lambda i: (0, 0)),
            pl.BlockSpec((n, c_pad), lambda i: (0, 0)),
        ],
        out_specs=pl.BlockSpec((1, 1, 128), lambda i: (i, 0, 0)),
        compiler_params=pltpu.CompilerParams(
            dimension_semantics=("arbitrary",),
        ),
        name="dtsh_loss",
    )(u_loc, y_loc, u_all, y_all)


def kernel(u, y):
    n, bit = u.shape
    c = y.shape[1]
    c_pad = ((c + 127) // 128) * 128
    y_p = jnp.pad(y, ((0, 0), (0, c_pad - c)))

    devs = jax.devices()
    n_shards = 2 if (len(devs) >= 2 and n % (2 * _BR) == 0) else 1
    if n_shards > 1:
        mesh = Mesh(np.array(devs[:n_shards]), ("c",))
        parts = jax.shard_map(
            _row_block_call,
            mesh=mesh,
            in_specs=(P("c", None), P("c", None), P(None, None), P(None, None)),
            out_specs=P("c", None, None),
            check_vma=False,
        )(u, y_p, u, y_p)
    else:
        parts = _row_block_call(u, y_p, u, y_p)

    sums = jnp.sum(parts[:, 0, :], axis=0)
    loss_sum, count, q_sum = sums[0], sums[1], sums[2]
    loss1 = jnp.where(
        count > 0, loss_sum / jnp.maximum(count, 1.0),
        jnp.asarray(0.0, u.dtype))
    loss2 = _LAM * q_sum / (n * bit)
    return loss1 + loss2


# speedup vs baseline: 3.0622x; 3.0622x over previous
import jax
import jax.numpy as jnp
from jax.experimental import pallas as pl
from jax.experimental.pallas import tpu as pltpu

_ALPHA = 5.0
_LAM = 1.0
_BR = 8
_L2E = 1.4426950408889634
_LN2 = 0.6931471805599453
_HI = 50.0 * _L2E
_LO = -100.0 * _L2E
_BIG = 1.0e9


def _dtsh_body(u_ref, y_ref, out_ref):
    i = pl.program_id(0)

    u_blk = u_ref[pl.ds(i * _BR, _BR), :]
    y_blk = y_ref[pl.ds(i * _BR, _BR), :]

    ip = jax.lax.dot_general(
        u_blk, u_ref[...], (((1,), (1,)), ((), ())),
        preferred_element_type=jnp.float32,
        precision=jax.lax.Precision.HIGHEST,
    )
    sim = jax.lax.dot_general(
        y_blk, y_ref[...], (((1,), (1,)), ((), ())),
        preferred_element_type=jnp.float32,
        precision=jax.lax.Precision.HIGHEST,
    )
    pos = sim > 0
    npos = jnp.sum(sim, axis=1)
    nneg = u_ref.shape[0] - npos

    a = jnp.where(pos, ip * _L2E, _BIG)
    c = jnp.where(pos, -_BIG, ip * _L2E + (_ALPHA * _L2E))

    s = c[:, None, :] - a[:, :, None]
    sc = jnp.clip(s, -_HI, 127.0)
    f = jnp.log(1.0 + jnp.exp2(sc))

    num = jnp.sum(f, axis=(1, 2))
    pair_count = jnp.maximum(npos * nneg, 1.0)
    row_loss = num / pair_count
    valid = (npos > 0.0) & (nneg > 0.0)
    contrib = jnp.sum(jnp.where(valid, row_loss, 0.0))
    vcount = jnp.sum(valid.astype(jnp.float32))

    q = jnp.sum((u_blk - jnp.sign(u_blk)) ** 2)

    lane = jax.lax.broadcasted_iota(jnp.int32, (1, 1, 128), 2)
    vals = jnp.where(
        lane == 0, contrib,
        jnp.where(lane == 1, vcount, jnp.where(lane == 2, q, 0.0)))
    out_ref[...] = vals


def kernel(u, y):
    n, bit = u.shape
    c = y.shape[1]
    c_pad = ((c + 127) // 128) * 128
    y_p = jnp.pad(y, ((0, 0), (0, c_pad - c)))
    g = n // _BR

    parts = pl.pallas_call(
        _dtsh_body,
        out_shape=jax.ShapeDtypeStruct((g, 1, 128), jnp.float32),
        grid=(g,),
        in_specs=[
            pl.BlockSpec((n, bit), lambda i: (0, 0)),
            pl.BlockSpec((n, c_pad), lambda i: (0, 0)),
        ],
        out_specs=pl.BlockSpec((1, 1, 128), lambda i: (i, 0, 0)),
        compiler_params=pltpu.CompilerParams(
            dimension_semantics=("arbitrary",),
        ),
        name="dtsh_loss",
    )(u, y_p)

    sums = jnp.sum(parts[:, 0, :], axis=0)
    loss_sum, count, q_sum = sums[0], sums[1], sums[2]
    loss1 = jnp.where(
        count > 0, loss_sum / jnp.maximum(count, 1.0),
        jnp.asarray(0.0, u.dtype))
    loss2 = _LAM * q_sum / (n * bit)
    return loss1 + loss2


# speedup vs baseline: 3.9383x; 1.2861x over previous
import jax
import jax.numpy as jnp
from jax.experimental import pallas as pl
from jax.experimental.pallas import tpu as pltpu

_ALPHA = 5.0
_LAM = 1.0
_BR = 32
_L2E = 1.4426950408889634
_LN2 = 0.6931471805599453
_HI = 50.0 * _L2E
_LO = -100.0 * _L2E
_BIG = 1.0e9


def _dtsh_body(u_ref, y_ref, out_ref):
    i = pl.program_id(0)

    u_blk = u_ref[pl.ds(i * _BR, _BR), :]
    y_blk = y_ref[pl.ds(i * _BR, _BR), :]

    ip = jax.lax.dot_general(
        u_blk, u_ref[...], (((1,), (1,)), ((), ())),
        preferred_element_type=jnp.float32,
        precision=jax.lax.Precision.HIGHEST,
    )
    sim = jax.lax.dot_general(
        y_blk, y_ref[...], (((1,), (1,)), ((), ())),
        preferred_element_type=jnp.float32,
        precision=jax.lax.Precision.HIGHEST,
    )
    pos = sim > 0
    npos = jnp.sum(sim, axis=1)
    nneg = u_ref.shape[0] - npos

    a = jnp.where(pos, ip * _L2E, _BIG)
    c = jnp.where(pos, -_BIG, ip * _L2E + (_ALPHA * _L2E))

    s = c[:, None, :] - a[:, :, None]
    sc = jnp.clip(s, -_HI, 127.0)
    f = jnp.log(1.0 + jnp.exp2(sc))

    num = jnp.sum(f, axis=(1, 2))
    pair_count = jnp.maximum(npos * nneg, 1.0)
    row_loss = num / pair_count
    valid = (npos > 0.0) & (nneg > 0.0)
    contrib = jnp.sum(jnp.where(valid, row_loss, 0.0))
    vcount = jnp.sum(valid.astype(jnp.float32))

    q = jnp.sum((u_blk - jnp.sign(u_blk)) ** 2)

    lane = jax.lax.broadcasted_iota(jnp.int32, (1, 1, 128), 2)
    vals = jnp.where(
        lane == 0, contrib,
        jnp.where(lane == 1, vcount, jnp.where(lane == 2, q, 0.0)))
    out_ref[...] = vals


def kernel(u, y):
    n, bit = u.shape
    c = y.shape[1]
    c_pad = ((c + 127) // 128) * 128
    y_p = jnp.pad(y, ((0, 0), (0, c_pad - c)))
    g = n // _BR

    parts = pl.pallas_call(
        _dtsh_body,
        out_shape=jax.ShapeDtypeStruct((g, 1, 128), jnp.float32),
        grid=(g,),
        in_specs=[
            pl.BlockSpec((n, bit), lambda i: (0, 0)),
            pl.BlockSpec((n, c_pad), lambda i: (0, 0)),
        ],
        out_specs=pl.BlockSpec((1, 1, 128), lambda i: (i, 0, 0)),
        compiler_params=pltpu.CompilerParams(
            dimension_semantics=("arbitrary",),
        ),
        name="dtsh_loss",
    )(u, y_p)

    sums = jnp.sum(parts[:, 0, :], axis=0)
    loss_sum, count, q_sum = sums[0], sums[1], sums[2]
    loss1 = jnp.where(
        count > 0, loss_sum / jnp.maximum(count, 1.0),
        jnp.asarray(0.0, u.dtype))
    loss2 = _LAM * q_sum / (n * bit)
    return loss1 + loss2


# speedup vs baseline: 3.9918x; 1.0136x over previous
import jax
import jax.numpy as jnp
from jax.experimental import pallas as pl
from jax.experimental.pallas import tpu as pltpu

_ALPHA = 5.0
_LAM = 1.0
_BR = 32
_L2E = 1.4426950408889634
_LN2 = 0.6931471805599453
_HI = 50.0 * _L2E
_LO = -100.0 * _L2E
_BIG = 1.0e9


def _dtsh_body(u_ref, y_ref, sel_ref, out_ref):
    i = pl.program_id(0)

    u_blk = u_ref[pl.ds(i * _BR, _BR), :]
    y_blk = y_ref[pl.ds(i * _BR, _BR), :]

    ip = jax.lax.dot_general(
        u_blk, u_ref[...], (((1,), (1,)), ((), ())),
        preferred_element_type=jnp.float32,
        precision=jax.lax.Precision.HIGHEST,
    )
    sim = jax.lax.dot_general(
        y_blk, y_ref[...], (((1,), (1,)), ((), ())),
        preferred_element_type=jnp.float32,
        precision=jax.lax.Precision.HIGHEST,
    )
    pos = sim > 0
    npos = jnp.sum(sim, axis=1)
    nneg = u_ref.shape[0] - npos

    a = jnp.where(pos, ip * _L2E, _BIG)
    c = jnp.where(pos, -_BIG, ip * _L2E + (_ALPHA * _L2E))

    s = c[:, None, :] - a[:, :, None]
    sb = s.astype(jnp.bfloat16)
    sc = jnp.clip(sb, jnp.bfloat16(-72.0), jnp.bfloat16(127.0))
    f = jnp.log(jnp.bfloat16(1.0) + jnp.exp2(sc))

    n = u_ref.shape[0]
    colsum = jax.lax.dot_general(
        sel_ref[...], f.reshape(_BR * n, n), (((1,), (0,)), ((), ())),
        preferred_element_type=jnp.float32,
    )
    num = jnp.sum(colsum, axis=1)
    pair_count = jnp.maximum(npos * nneg, 1.0)
    row_loss = num / pair_count
    valid = (npos > 0.0) & (nneg > 0.0)
    contrib = jnp.sum(jnp.where(valid, row_loss, 0.0))
    vcount = jnp.sum(valid.astype(jnp.float32))

    q = jnp.sum((u_blk - jnp.sign(u_blk)) ** 2)

    lane = jax.lax.broadcasted_iota(jnp.int32, (1, 1, 128), 2)
    vals = jnp.where(
        lane == 0, contrib,
        jnp.where(lane == 1, vcount, jnp.where(lane == 2, q, 0.0)))
    out_ref[...] = vals


def kernel(u, y):
    n, bit = u.shape
    c = y.shape[1]
    c_pad = ((c + 127) // 128) * 128
    y_p = jnp.pad(y, ((0, 0), (0, c_pad - c)))
    g = n // _BR
    sel = jnp.repeat(jnp.eye(_BR, dtype=jnp.bfloat16), n, axis=1)

    parts = pl.pallas_call(
        _dtsh_body,
        out_shape=jax.ShapeDtypeStruct((g, 1, 128), jnp.float32),
        grid=(g,),
        in_specs=[
            pl.BlockSpec((n, bit), lambda i: (0, 0)),
            pl.BlockSpec((n, c_pad), lambda i: (0, 0)),
            pl.BlockSpec((_BR, _BR * n), lambda i: (0, 0)),
        ],
        out_specs=pl.BlockSpec((1, 1, 128), lambda i: (i, 0, 0)),
        compiler_params=pltpu.CompilerParams(
            dimension_semantics=("arbitrary",),
        ),
        name="dtsh_loss",
    )(u, y_p, sel)

    sums = jnp.sum(parts[:, 0, :], axis=0)
    loss_sum, count, q_sum = sums[0], sums[1], sums[2]
    loss1 = jnp.where(
        count > 0, loss_sum / jnp.maximum(count, 1.0),
        jnp.asarray(0.0, u.dtype))
    loss2 = _LAM * q_sum / (n * bit)
    return loss1 + loss2


# speedup vs baseline: 4.0131x; 1.0053x over previous
import jax
import jax.numpy as jnp
from jax.experimental import pallas as pl
from jax.experimental.pallas import tpu as pltpu

_ALPHA = 5.0
_LAM = 1.0
_BR = 64
_L2E = 1.4426950408889634
_LN2 = 0.6931471805599453
_HI = 50.0 * _L2E
_LO = -100.0 * _L2E
_BIG = 1.0e9


def _dtsh_body(u_ref, y_ref, sel_ref, out_ref):
    i = pl.program_id(0)

    u_blk = u_ref[pl.ds(i * _BR, _BR), :]
    y_blk = y_ref[pl.ds(i * _BR, _BR), :]

    ip = jax.lax.dot_general(
        u_blk, u_ref[...], (((1,), (1,)), ((), ())),
        preferred_element_type=jnp.float32,
        precision=jax.lax.Precision.HIGHEST,
    )
    sim = jax.lax.dot_general(
        y_blk, y_ref[...], (((1,), (1,)), ((), ())),
        preferred_element_type=jnp.float32,
        precision=jax.lax.Precision.HIGHEST,
    )
    pos = sim > 0
    npos = jnp.sum(sim, axis=1)
    nneg = u_ref.shape[0] - npos

    a = jnp.where(pos, ip * _L2E, _BIG)
    c = jnp.where(pos, -_BIG, ip * _L2E + (_ALPHA * _L2E))

    s = c[:, None, :] - a[:, :, None]
    sb = s.astype(jnp.bfloat16)
    sc = jnp.clip(sb, jnp.bfloat16(-72.0), jnp.bfloat16(127.0))
    f = jnp.log(jnp.bfloat16(1.0) + jnp.exp2(sc))

    n = u_ref.shape[0]
    colsum = jax.lax.dot_general(
        sel_ref[...], f.reshape(_BR * n, n), (((1,), (0,)), ((), ())),
        preferred_element_type=jnp.float32,
    )
    num = jnp.sum(colsum, axis=1)
    pair_count = jnp.maximum(npos * nneg, 1.0)
    row_loss = num / pair_count
    valid = (npos > 0.0) & (nneg > 0.0)
    contrib = jnp.sum(jnp.where(valid, row_loss, 0.0))
    vcount = jnp.sum(valid.astype(jnp.float32))

    q = jnp.sum((u_blk - jnp.sign(u_blk)) ** 2)

    lane = jax.lax.broadcasted_iota(jnp.int32, (1, 1, 128), 2)
    vals = jnp.where(
        lane == 0, contrib,
        jnp.where(lane == 1, vcount, jnp.where(lane == 2, q, 0.0)))
    out_ref[...] = vals


def kernel(u, y):
    n, bit = u.shape
    c = y.shape[1]
    c_pad = ((c + 127) // 128) * 128
    y_p = jnp.pad(y, ((0, 0), (0, c_pad - c)))
    g = n // _BR
    sel = jnp.repeat(jnp.eye(_BR, dtype=jnp.bfloat16), n, axis=1)

    parts = pl.pallas_call(
        _dtsh_body,
        out_shape=jax.ShapeDtypeStruct((g, 1, 128), jnp.float32),
        grid=(g,),
        in_specs=[
            pl.BlockSpec((n, bit), lambda i: (0, 0)),
            pl.BlockSpec((n, c_pad), lambda i: (0, 0)),
            pl.BlockSpec((_BR, _BR * n), lambda i: (0, 0)),
        ],
        out_specs=pl.BlockSpec((1, 1, 128), lambda i: (i, 0, 0)),
        compiler_params=pltpu.CompilerParams(
            dimension_semantics=("arbitrary",),
        ),
        name="dtsh_loss",
    )(u, y_p, sel)

    sums = jnp.sum(parts[:, 0, :], axis=0)
    loss_sum, count, q_sum = sums[0], sums[1], sums[2]
    loss1 = jnp.where(
        count > 0, loss_sum / jnp.maximum(count, 1.0),
        jnp.asarray(0.0, u.dtype))
    loss2 = _LAM * q_sum / (n * bit)
    return loss1 + loss2
